# baseline (speedup 1.0000x reference)
import functools

import jax
import jax.numpy as jnp
from jax import lax
from jax.experimental import pallas as pl
from jax.experimental.pallas import tpu as pltpu
from jax.experimental.pallas import tpu_sc as plsc

B, N, D = 4, 8192, 2048
NC, NS, L = 2, 16, 16
NW = NC * NS
COLS_PER_W = (B * D) // NW
CW = 128
G = CW // L
NCC = COLS_PER_W // CW
RB = 256
NRC = N // RB
UNROLL = 4

_mesh = plsc.VectorSubcoreMesh(core_axis_name="c", subcore_axis_name="s")


@functools.partial(
    pl.kernel,
    out_type=jax.ShapeDtypeStruct((B * D,), jnp.int32),
    mesh=_mesh,
    scratch_types=[
        pltpu.VMEM((RB, CW), jnp.float32),
        pltpu.VMEM((COLS_PER_W,), jnp.int32),
    ],
)
def _argmin_sc(x_hbm, out_hbm, buf, outv):
    wid = lax.axis_index("s") * NC + lax.axis_index("c")
    base = wid * COLS_PER_W
    b = base // D
    j0 = base % D

    for cc in range(NCC):
        jc = j0 + cc * CW

        def rc_body(rc, carry):
            r0 = rc * RB
            pltpu.sync_copy(x_hbm.at[b, pl.ds(r0, RB), pl.ds(jc, CW)], buf)

            def row_body(r, carry2):
                mv, mi = carry2
                rv = jnp.full((L,), r0 + r, jnp.int32)
                mv2, mi2 = [], []
                for g in range(G):
                    v = buf[r, g * L:(g + 1) * L]
                    p = v < mv[g]
                    mv2.append(jnp.where(p, v, mv[g]))
                    mi2.append(jnp.where(p, rv, mi[g]))
                return (tuple(mv2), tuple(mi2))

            return lax.fori_loop(0, RB, row_body, carry, unroll=UNROLL)

        init = (
            tuple(jnp.full((L,), jnp.inf, jnp.float32) for _ in range(G)),
            tuple(jnp.zeros((L,), jnp.int32) for _ in range(G)),
        )
        _, minis = lax.fori_loop(0, NRC, rc_body, init)
        for g in range(G):
            outv[cc * CW + g * L:cc * CW + (g + 1) * L] = minis[g]

    pltpu.sync_copy(outv, out_hbm.at[pl.ds(base, COLS_PER_W)])


def kernel(x):
    out = _argmin_sc(x)
    return out.reshape(B, D).astype(jnp.int64)


# speedup vs baseline: 1.6522x; 1.6522x over previous
import functools

import jax
import jax.numpy as jnp
from jax import lax
from jax.experimental import pallas as pl
from jax.experimental.pallas import tpu as pltpu
from jax.experimental.pallas import tpu_sc as plsc

B, N, D = 4, 8192, 2048
NC, NS, L = 2, 16, 16
NW = NC * NS
COLS_PER_W = (B * D) // NW
CW = 128
G = CW // L
NCC = COLS_PER_W // CW
RB = 256
NRC = N // RB
UNROLL = 4

_mesh = plsc.VectorSubcoreMesh(core_axis_name="c", subcore_axis_name="s")


@functools.partial(
    pl.kernel,
    out_type=jax.ShapeDtypeStruct((B * D,), jnp.int32),
    mesh=_mesh,
    scratch_types=[
        pltpu.VMEM((RB, CW), jnp.float32),
        pltpu.VMEM((RB, CW), jnp.float32),
        pltpu.VMEM((COLS_PER_W,), jnp.int32),
        pltpu.SemaphoreType.DMA,
        pltpu.SemaphoreType.DMA,
    ],
)
def _argmin_sc(x_hbm, out_hbm, buf0, buf1, outv, sem0, sem1):
    wid = lax.axis_index("s") * NC + lax.axis_index("c")
    base = wid * COLS_PER_W
    b = base // D
    j0 = base % D

    bufs = (buf0, buf1)
    sems = (sem0, sem1)

    for cc in range(NCC):
        jc = j0 + cc * CW

        def copy(rc, ph):
            return pltpu.make_async_copy(
                x_hbm.at[b, pl.ds(rc * RB, RB), pl.ds(jc, CW)],
                bufs[ph], sems[ph])

        def compute(buf, r0, carry):
            def row_body(r, carry2):
                mv, mi = carry2
                rv = jnp.full((L,), r0 + r, jnp.int32)
                mv2, mi2 = [], []
                for g in range(G):
                    v = buf[r, g * L:(g + 1) * L]
                    p = v < mv[g]
                    mv2.append(jnp.where(p, v, mv[g]))
                    mi2.append(jnp.where(p, rv, mi[g]))
                return (tuple(mv2), tuple(mi2))

            return lax.fori_loop(0, RB, row_body, carry, unroll=UNROLL)

        copy(0, 0).start()

        def pair_body(i, carry):
            rc0 = 2 * i
            copy(rc0 + 1, 1).start()
            copy(rc0, 0).wait()
            carry = compute(buf0, rc0 * RB, carry)

            @pl.when(rc0 + 2 < NRC)
            def _():
                copy(rc0 + 2, 0).start()

            copy(rc0 + 1, 1).wait()
            carry = compute(buf1, (rc0 + 1) * RB, carry)
            return carry

        init = (
            tuple(jnp.full((L,), jnp.inf, jnp.float32) for _ in range(G)),
            tuple(jnp.zeros((L,), jnp.int32) for _ in range(G)),
        )
        _, minis = lax.fori_loop(0, NRC // 2, pair_body, init)
        for g in range(G):
            outv[cc * CW + g * L:cc * CW + (g + 1) * L] = minis[g]

    pltpu.sync_copy(outv, out_hbm.at[pl.ds(base, COLS_PER_W)])


def kernel(x):
    out = _argmin_sc(x)
    return out.reshape(B, D).astype(jnp.int64)


# speedup vs baseline: 2.3738x; 1.4367x over previous
import functools

import jax
import jax.numpy as jnp
from jax import lax
from jax.experimental import pallas as pl
from jax.experimental.pallas import tpu as pltpu
from jax.experimental.pallas import tpu_sc as plsc

B, N, D = 4, 8192, 2048
NC, NS, L = 2, 16, 16
NW = NC * NS
COLS_PER_W = (B * D) // NW
CW = 128
G = CW // L
NCC = COLS_PER_W // CW
RB = 256
NRC = N // RB
UNROLL = 4

_mesh = plsc.VectorSubcoreMesh(core_axis_name="c", subcore_axis_name="s")


@functools.partial(
    pl.kernel,
    out_type=jax.ShapeDtypeStruct((B * D,), jnp.int32),
    mesh=_mesh,
    scratch_types=[
        pltpu.VMEM((RB, CW), jnp.float32),
        pltpu.VMEM((RB, CW), jnp.float32),
        pltpu.VMEM((COLS_PER_W,), jnp.int32),
        pltpu.SemaphoreType.DMA,
        pltpu.SemaphoreType.DMA,
    ],
)
def _argmin_sc(x_hbm, out_hbm, buf0, buf1, outv, sem0, sem1):
    wid = lax.axis_index("s") * NC + lax.axis_index("c")
    base = wid * COLS_PER_W
    b = base // D
    j0 = base % D

    bufs = (buf0, buf1)
    sems = (sem0, sem1)

    for cc in range(NCC):
        jc = j0 + cc * CW

        def copy(rc, ph):
            return pltpu.make_async_copy(
                x_hbm.at[b, pl.ds(rc * RB, RB), pl.ds(jc, CW)],
                bufs[ph], sems[ph])

        def compute(buf, r0, carry):
            def row_body(r, carry2):
                mv, mi = carry2
                rv = jnp.full((L,), r0 + r, jnp.int32)
                mv2, mi2 = [], []
                for g in range(G):
                    v = buf[r, g * L:(g + 1) * L]
                    p = v < mv[g]
                    mv2.append(jnp.minimum(v, mv[g]))
                    mi2.append(jnp.where(p, rv, mi[g]))
                return (tuple(mv2), tuple(mi2))

            return lax.fori_loop(0, RB, row_body, carry, unroll=UNROLL)

        copy(0, 0).start()

        def pair_body(i, carry):
            rc0 = 2 * i
            copy(rc0 + 1, 1).start()
            copy(rc0, 0).wait()
            carry = compute(buf0, rc0 * RB, carry)

            @pl.when(rc0 + 2 < NRC)
            def _():
                copy(rc0 + 2, 0).start()

            copy(rc0 + 1, 1).wait()
            carry = compute(buf1, (rc0 + 1) * RB, carry)
            return carry

        init = (
            tuple(jnp.full((L,), jnp.inf, jnp.float32) for _ in range(G)),
            tuple(jnp.zeros((L,), jnp.int32) for _ in range(G)),
        )
        _, minis = lax.fori_loop(0, NRC // 2, pair_body, init)
        for g in range(G):
            outv[cc * CW + g * L:cc * CW + (g + 1) * L] = minis[g]

    pltpu.sync_copy(outv, out_hbm.at[pl.ds(base, COLS_PER_W)])


def kernel(x):
    out = _argmin_sc(x)
    return out.reshape(B, D).astype(jnp.int64)
